# Initial kernel scaffold; baseline (speedup 1.0000x reference)
import jax
import jax.numpy as jnp
from jax.experimental import pallas as pl


def kernel(x, edge_index, edge_label, weight, bias, trans_weight):
    raise NotImplementedError("write your pallas kernel here")


# speedup vs baseline: 4.0647x; 4.0647x over previous
import functools

import jax
import jax.numpy as jnp
from jax import lax
from jax.experimental import pallas as pl
from jax.experimental.pallas import tpu as pltpu
from jax.experimental.pallas import tpu_sc as plsc

N_NODES = 10000
D_FEAT = 128
LANES = 128
BLK = 8
CHUNK = LANES
EBLK = BLK * CHUNK
N_TILES = 16
N_CORES = 2
ACC_ROWS = 10240
ZPT = ACC_ROWS // N_TILES


def _sc_aggregate(x, srcs, dsts, el8, zm, ept):
  nblocks = ept // EBLK
  rows_pt = ept // LANES

  mesh = plsc.VectorSubcoreMesh(core_axis_name="c", subcore_axis_name="s")

  @functools.partial(
      pl.kernel,
      out_type=(
          jax.ShapeDtypeStruct((N_CORES, ACC_ROWS, D_FEAT), jnp.float32),
          jax.ShapeDtypeStruct((N_CORES, ACC_ROWS, 8), jnp.float32),
      ),
      mesh=mesh,
      scratch_types=[
          pltpu.VMEM_SHARED((ACC_ROWS, D_FEAT), jnp.float32),
          pltpu.VMEM_SHARED((ACC_ROWS, 8), jnp.float32),
          pltpu.VMEM((BLK, LANES), jnp.int32),
          pltpu.VMEM((BLK, LANES), jnp.int32),
          pltpu.VMEM((2, CHUNK, D_FEAT), jnp.float32),
          pltpu.VMEM((EBLK, 8), jnp.float32),
          pltpu.SemaphoreType.DMA,
          pltpu.SemaphoreType.DMA,
          pltpu.SemaphoreType.DMA,
          pltpu.SemaphoreType.DMA,
          pltpu.SemaphoreType.DMA,
      ],
      compiler_params=pltpu.CompilerParams(use_tc_tiling_on_sc=False),
  )
  def sck(x_hbm, srcs_hbm, dsts_hbm, el8_hbm, zm_hbm, accx_out, accm_out,
          accx_s, accm_s, sidx, didx, xbuf, lbuf,
          gsem0, gsem1, xsem0, xsem1, msem):
    c = lax.axis_index("c")
    s = lax.axis_index("s")
    gsem = (gsem0, gsem1)
    xsem = (xsem0, xsem1)

    def zbody(i, _):
      for j in range(D_FEAT // 16):
        xbuf[0, i, pl.ds(j * 16, 16)] = jnp.zeros((16,), jnp.float32)
      return _
    lax.fori_loop(0, CHUNK, zbody, 0)

    zr = s * ZPT
    for kk in range(ZPT // CHUNK):
      pltpu.sync_copy(xbuf.at[0], accx_s.at[pl.ds(zr + kk * CHUNK, CHUNK)])
    pltpu.sync_copy(zm_hbm.at[pl.ds(zr, ZPT)], accm_s.at[pl.ds(zr, ZPT)])
    plsc.subcore_barrier()

    def block(blk, _):
      rb = s * rows_pt + blk * BLK
      pltpu.sync_copy(srcs_hbm.at[c, pl.ds(rb, BLK)], sidx)
      pltpu.sync_copy(dsts_hbm.at[c, pl.ds(rb, BLK)], didx)
      pltpu.sync_copy(el8_hbm.at[pl.ds(rb * LANES, EBLK)], lbuf)
      for k in range(BLK):
        p = k % 2
        if k >= 2:
          pltpu.make_async_copy(
              xbuf.at[p], accx_s.at[didx.at[k - 2]], xsem[p]).wait()
        pltpu.async_copy(x_hbm.at[sidx.at[k]], xbuf.at[p], gsem[p])
        if k >= 1:
          q = 1 - p
          pltpu.make_async_copy(x_hbm.at[sidx.at[k - 1]], xbuf.at[q],
                                gsem[q]).wait()
          pltpu.async_copy(xbuf.at[q], accx_s.at[didx.at[k - 1]],
                           xsem[q], add=True)
          pltpu.async_copy(lbuf.at[pl.ds((k - 1) * CHUNK, CHUNK)],
                           accm_s.at[didx.at[k - 1]], msem, add=True)
      pl_ = (BLK - 1) % 2
      pltpu.make_async_copy(x_hbm.at[sidx.at[BLK - 1]], xbuf.at[pl_],
                            gsem[pl_]).wait()
      pltpu.async_copy(xbuf.at[pl_], accx_s.at[didx.at[BLK - 1]],
                       xsem[pl_], add=True)
      pltpu.async_copy(lbuf.at[pl.ds((BLK - 1) * CHUNK, CHUNK)],
                       accm_s.at[didx.at[BLK - 1]], msem, add=True)
      for p in range(2):
        pltpu.make_async_copy(
            xbuf.at[p], accx_s.at[didx.at[0]], xsem[p]).wait()
      for k in range(BLK):
        pltpu.make_async_copy(lbuf.at[pl.ds(0, CHUNK)],
                              accm_s.at[didx.at[0]], msem).wait()
      return _
    lax.fori_loop(0, nblocks, block, 0)
    plsc.subcore_barrier()

    orows = ACC_ROWS // N_TILES
    orb = s * orows
    pltpu.sync_copy(accx_s.at[pl.ds(orb, orows)],
                    accx_out.at[c, pl.ds(orb, orows)])
    pltpu.sync_copy(accm_s.at[pl.ds(orb, orows)],
                    accm_out.at[c, pl.ds(orb, orows)])

  return sck(x, srcs, dsts, el8, zm)


def _tc_epilogue(ax0, ax1, am0, am1, weight, trans_weight, bias2d):
  blk = 1000
  grid = (N_NODES // blk,)

  def body(ax0_r, ax1_r, am0_r, am1_r, w_r, tw_r, b_r, o_r):
    w = w_r[...]
    tw = tw_r[...]
    f32 = jnp.float32
    a = jnp.dot(ax0_r[...], w[0:128], preferred_element_type=f32)
    a = a + jnp.dot(am0_r[:, 0:4], jnp.dot(tw, w[128:256],
                                           preferred_element_type=f32),
                    preferred_element_type=f32)
    c0 = jnp.maximum(am0_r[:, 4:5], 1.0)
    b = jnp.dot(ax1_r[...], w[256:384], preferred_element_type=f32)
    b = b + jnp.dot(am1_r[:, 0:4], jnp.dot(tw, w[384:512],
                                           preferred_element_type=f32),
                    preferred_element_type=f32)
    c1 = jnp.maximum(am1_r[:, 4:5], 1.0)
    o_r[...] = a / c0 + b / c1 + b_r[...]

  return pl.pallas_call(
      body,
      grid=grid,
      in_specs=[
          pl.BlockSpec((blk, D_FEAT), lambda i: (i, 0)),
          pl.BlockSpec((blk, D_FEAT), lambda i: (i, 0)),
          pl.BlockSpec((blk, 8), lambda i: (i, 0)),
          pl.BlockSpec((blk, 8), lambda i: (i, 0)),
          pl.BlockSpec((512, 128), lambda i: (0, 0)),
          pl.BlockSpec((4, 128), lambda i: (0, 0)),
          pl.BlockSpec((1, 128), lambda i: (0, 0)),
      ],
      out_specs=pl.BlockSpec((blk, D_FEAT), lambda i: (i, 0)),
      out_shape=jax.ShapeDtypeStruct((N_NODES, D_FEAT), jnp.float32),
  )(ax0, ax1, am0, am1, weight, trans_weight, bias2d)


def kernel(x, edge_index, edge_label, weight, bias, trans_weight):
  e = edge_index.shape[1]
  ept = ((e + N_TILES * EBLK - 1) // (N_TILES * EBLK)) * EBLK
  e_pad = ept * N_TILES
  pad = e_pad - e

  ei = edge_index.astype(jnp.int32)
  row_s = jnp.concatenate([ei[0], jnp.zeros((pad,), jnp.int32)])
  col_s = jnp.concatenate([ei[1], jnp.zeros((pad,), jnp.int32)])
  row_d = jnp.concatenate([ei[0], jnp.full((pad,), N_NODES, jnp.int32)])
  col_d = jnp.concatenate([ei[1], jnp.full((pad,), N_NODES, jnp.int32)])
  srcs = jnp.stack([col_s, row_s]).reshape(2, e_pad // LANES, LANES)
  dsts = jnp.stack([row_d, col_d]).reshape(2, e_pad // LANES, LANES)
  el8 = jnp.concatenate(
      [edge_label, jnp.ones((e, 1), jnp.float32),
       jnp.zeros((e, 3), jnp.float32)], axis=1)
  el8 = jnp.concatenate([el8, jnp.zeros((pad, 8), jnp.float32)], axis=0)
  zm = jnp.zeros((ACC_ROWS, 8), jnp.float32)

  acc_x, acc_m = _sc_aggregate(x, srcs, dsts, el8, zm, ept)

  return _tc_epilogue(
      acc_x[0, :N_NODES], acc_x[1, :N_NODES],
      acc_m[0, :N_NODES], acc_m[1, :N_NODES],
      weight, trans_weight, bias.reshape(1, D_FEAT))
